# baseline (speedup 1.0000x reference)
import math

import jax
import jax.numpy as jnp
from jax.experimental import pallas as pl
from jax.experimental.pallas import tpu as pltpu

_POS = 1
_NEG = 0
_ALPHA = 0.5
_GAMMA = 1.5
_BB = math.e ** (_GAMMA / _ALPHA) - 1.0


def _stats_body(gtc_ref, pc_ref, pb_ref, gb_ref, negce_ref, stats_ref):
    i = pl.program_id(0)
    g = gtc_ref[0, 0, :]
    x = pc_ref[...]

    mx = jnp.max(x, axis=1)
    lse = mx + jnp.log(jnp.sum(jnp.exp(x - mx[:, None]), axis=1))
    cls_iota = jax.lax.broadcasted_iota(jnp.int32, x.shape, 1)
    picked = jnp.sum(jnp.where(cls_iota == g[:, None], x, 0.0), axis=1)
    ce = lse - picked

    posm = g == _POS
    negm = g == _NEG
    negce_ref[0, 0, :] = jnp.where(negm, jnp.maximum(ce, 0.0), 0.0)

    d = jnp.abs(pb_ref[...] - gb_ref[...])
    bl = jnp.where(
        d < 1.0,
        _ALPHA / _BB * (_BB * d + 1.0) * jnp.log(_BB * d + 1.0) - _ALPHA * d,
        _GAMMA * d + _GAMMA / _BB - _ALPHA,
    )
    rs = jnp.sum(bl, axis=1)

    posf = posm.astype(jnp.float32)
    s0 = jnp.sum(posf)
    s1 = jnp.sum(negm.astype(jnp.float32))
    s2 = jnp.sum(jnp.where(posm, ce, 0.0))
    s3 = jnp.sum(posf * rs)

    @pl.when(i == 0)
    def _init():
        stats_ref[0] = s0
        stats_ref[1] = s1
        stats_ref[2] = s2
        stats_ref[3] = s3

    @pl.when(i != 0)
    def _acc():
        stats_ref[0] += s0
        stats_ref[1] += s1
        stats_ref[2] += s2
        stats_ref[3] += s3


def _select_body(stats_ref, neg_ref, out_ref):
    pos_cnt = stats_ref[0]
    neg_cnt = stats_ref[1]
    cls_pos = stats_ref[2]
    loc_sum = stats_ref[3]

    kf = jnp.minimum(pos_cnt, neg_cnt)
    k = kf.astype(jnp.int32)

    x = neg_ref[...]
    u = jax.lax.bitcast_convert_type(x, jnp.int32)

    def body(j, prefix):
        trial = prefix | (jnp.int32(1) << (jnp.int32(30) - j))
        cnt = jnp.sum((u >= trial).astype(jnp.int32))
        return jnp.where(cnt >= k, trial, prefix)

    t_bits = jax.lax.fori_loop(0, 31, body, jnp.int32(0))

    gt = u > t_bits
    cnt_gt = jnp.sum(gt.astype(jnp.int32))
    sum_gt = jnp.sum(jnp.where(gt, x, 0.0))
    t_val = jax.lax.bitcast_convert_type(t_bits, jnp.float32)
    cls_neg = jnp.where(
        k > 0, sum_gt + (kf - cnt_gt.astype(jnp.float32)) * t_val, 0.0
    )

    has_pos = pos_cnt > 0.0
    ns = pos_cnt + kf
    out_ref[0] = jnp.where(has_pos, loc_sum / jnp.maximum(pos_cnt, 1.0), 0.0)
    out_ref[1] = jnp.where(
        has_pos, (cls_pos + cls_neg) / jnp.maximum(ns, 1.0), 0.0
    )


def kernel(predicted_boxes, predicted_classes, gt_bboxes, gt_classes):
    b, a, c = predicted_classes.shape
    n = b * a
    bs = 4000
    nblk = n // bs

    pc = predicted_classes.reshape(n, c)
    pb = predicted_boxes.reshape(n, 4)
    gb = gt_bboxes.reshape(n, 4)
    gtc = gt_classes.reshape(nblk, 1, bs)

    negce, stats = pl.pallas_call(
        _stats_body,
        grid=(nblk,),
        in_specs=[
            pl.BlockSpec((1, 1, bs), lambda i: (i, 0, 0)),
            pl.BlockSpec((bs, c), lambda i: (i, 0)),
            pl.BlockSpec((bs, 4), lambda i: (i, 0)),
            pl.BlockSpec((bs, 4), lambda i: (i, 0)),
        ],
        out_specs=[
            pl.BlockSpec((1, 1, bs), lambda i: (i, 0, 0)),
            pl.BlockSpec(memory_space=pltpu.SMEM),
        ],
        out_shape=[
            jax.ShapeDtypeStruct((nblk, 1, bs), jnp.float32),
            jax.ShapeDtypeStruct((4,), jnp.float32),
        ],
        compiler_params=pltpu.CompilerParams(
            dimension_semantics=("arbitrary",),
        ),
    )(gtc, pc, pb, gb)

    out = pl.pallas_call(
        _select_body,
        in_specs=[
            pl.BlockSpec(memory_space=pltpu.SMEM),
            pl.BlockSpec((n // 128, 128), lambda: (0, 0)),
        ],
        out_specs=pl.BlockSpec(memory_space=pltpu.SMEM),
        out_shape=jax.ShapeDtypeStruct((2,), jnp.float32),
    )(stats, negce.reshape(n // 128, 128))

    return (out[0], out[1])
